# baseline (speedup 1.0000x reference)
import jax
import jax.numpy as jnp
from jax.experimental import pallas as pl
from jax.experimental.pallas import tpu as pltpu

ROUTING_ITERS = 3
EPSILON = 1e-8


def _routing_kernel(x_ref, w_ref, o_ref, u_ref, xb_ref):
    C, IC, OC, R = w_ref.shape
    BB = x_ref.shape[0]

    for i in range(IC):
        xb_ref[i] = jnp.broadcast_to(x_ref[:, i, :][:, None, :], (BB, 8, R))

    usum = [None] * C
    for rt in range(R // 128):
        rsl = slice(rt * 128, (rt + 1) * 128)
        xch = []
        for i in range(IC):
            xc = xb_ref[i, :, :, rsl]
            xch.append(jnp.concatenate([xc, xc], axis=1))
        for c in range(C):
            acc = None
            for i in range(IC):
                term = xch[i] * w_ref[c, i, :, rsl][None, :, :]
                acc = term if acc is None else acc + term
            u_ref[c, :, :, rsl] = acc
            usum[c] = acc if usum[c] is None else usum[c] + acc

    logits = [None] * C
    for it in range(ROUTING_ITERS):
        if it == 0:
            coup = None
        else:
            es = [jnp.exp(logits[c]) for c in range(C)]
            den = es[0]
            for c in range(1, C):
                den = den + es[c]
            inv = 1.0 / den
            coup = [es[c] * inv for c in range(C)]

        for c in range(C):
            if coup is None:
                s_c = jnp.sum(usum[c], axis=-1) * (1.0 / C)
            else:
                s_c = jnp.sum(coup[c][:, None, :] * u_ref[c], axis=-1)
            nsq = jnp.sum(s_c * s_c, axis=-1, keepdims=True)
            v_c = s_c * (nsq / (1.0 + nsq) / (jnp.sqrt(nsq) + EPSILON))
            if it < ROUTING_ITERS - 1:
                agree = jnp.sum(u_ref[c] * v_c[:, :, None], axis=1)
                logits[c] = agree if logits[c] is None else logits[c] + agree
            else:
                o_ref[:, c, :] = v_c


def kernel(x, weights):
    B, R, IC = x.shape
    C, OC = weights.shape[2], weights.shape[4]
    BB = 8

    xt = x.transpose(0, 2, 1)
    wt = weights[0].transpose(1, 2, 3, 0)

    return pl.pallas_call(
        _routing_kernel,
        grid=(B // BB,),
        in_specs=[
            pl.BlockSpec((BB, IC, R), lambda i: (i, 0, 0)),
            pl.BlockSpec((C, IC, OC, R), lambda i: (0, 0, 0, 0)),
        ],
        out_specs=pl.BlockSpec((BB, C, OC), lambda i: (i, 0, 0)),
        out_shape=jax.ShapeDtypeStruct((B, C, OC), jnp.float32),
        scratch_shapes=[
            pltpu.VMEM((C, BB, OC, R), jnp.float32),
            pltpu.VMEM((IC, BB, 8, R), jnp.float32),
        ],
        compiler_params=pltpu.CompilerParams(
            dimension_semantics=("parallel",),
            vmem_limit_bytes=56 * 1024 * 1024,
        ),
    )(xt, wt)


# speedup vs baseline: 1.1373x; 1.1373x over previous
import jax
import jax.numpy as jnp
from jax.experimental import pallas as pl
from jax.experimental.pallas import tpu as pltpu

ROUTING_ITERS = 3
EPSILON = 1e-8


def _routing_kernel(x_ref, w_ref, o_ref, u_ref, xb_ref, b_ref):
    C, IC, OC, R = w_ref.shape
    BB = x_ref.shape[0]

    for i in range(IC):
        xb_ref[i] = jnp.broadcast_to(x_ref[:, i, :][:, None, :], (BB, 8, R))

    usum = [None] * C
    for rt in range(R // 128):
        rsl = slice(rt * 128, (rt + 1) * 128)
        xch = []
        for i in range(IC):
            xc = xb_ref[i, :, :, rsl]
            xch.append(jnp.concatenate([xc, xc], axis=1))
        for c in range(C):
            acc = None
            for i in range(IC):
                term = xch[i] * w_ref[c, i, :, rsl][None, :, :]
                acc = term if acc is None else acc + term
            u_ref[c, :, :, rsl] = acc
            usum[c] = acc if usum[c] is None else usum[c] + acc

    for it in range(ROUTING_ITERS):
        if it == 0:
            coup = None
        else:
            es = [jnp.exp(b_ref[c]) for c in range(C)]
            den = es[0]
            for c in range(1, C):
                den = den + es[c]
            inv = 1.0 / den
            coup = [es[c] * inv for c in range(C)]

        for c in range(C):
            if coup is None:
                s_c = jnp.sum(usum[c], axis=-1) * (1.0 / C)
            else:
                s_c = jnp.sum(coup[c][:, None, :] * u_ref[c], axis=-1)
            nsq = jnp.sum(s_c * s_c, axis=-1, keepdims=True)
            v_c = s_c * (nsq / (1.0 + nsq) / (jnp.sqrt(nsq) + EPSILON))
            if it < ROUTING_ITERS - 1:
                agree = jnp.sum(u_ref[c] * v_c[:, :, None], axis=1)
                b_ref[c] = agree if it == 0 else b_ref[c] + agree
            else:
                o_ref[:, c, :] = v_c


def kernel(x, weights):
    B, R, IC = x.shape
    C, OC = weights.shape[2], weights.shape[4]
    BB = 16

    xt = x.transpose(0, 2, 1)
    wt = weights[0].transpose(1, 2, 3, 0)

    return pl.pallas_call(
        _routing_kernel,
        grid=(B // BB,),
        in_specs=[
            pl.BlockSpec((BB, IC, R), lambda i: (i, 0, 0)),
            pl.BlockSpec((C, IC, OC, R), lambda i: (0, 0, 0, 0)),
        ],
        out_specs=pl.BlockSpec((BB, C, OC), lambda i: (i, 0, 0)),
        out_shape=jax.ShapeDtypeStruct((B, C, OC), jnp.float32),
        scratch_shapes=[
            pltpu.VMEM((C, BB, OC, R), jnp.float32),
            pltpu.VMEM((IC, BB, 8, R), jnp.float32),
            pltpu.VMEM((C, BB, R), jnp.float32),
        ],
        compiler_params=pltpu.CompilerParams(
            dimension_semantics=("parallel",),
            vmem_limit_bytes=56 * 1024 * 1024,
        ),
    )(xt, wt)
